# Initial kernel scaffold; baseline (speedup 1.0000x reference)
import jax
import jax.numpy as jnp
from jax.experimental import pallas as pl


def kernel(x, edge_index, W1, b1, g1, be1, W2, b2, g2, be2, Wf, bf):
    raise NotImplementedError("write your pallas kernel here")


# speedup vs baseline: 10.4094x; 10.4094x over previous
import functools

import jax
import jax.numpy as jnp
from jax import lax
from jax.experimental import pallas as pl
from jax.experimental.pallas import tpu as pltpu
from jax.experimental.pallas import tpu_sc as plsc

NC = 2
NS = 16
LANES = 128


def _mesh():
    return plsc.VectorSubcoreMesh(core_axis_name="c", subcore_axis_name="s",
                                  num_cores=NC, num_subcores=NS)


def _zero_fill(ref, n_words):
    def body(k, _):
        ref[pl.ds(k * 16, 16)] = jnp.zeros((16,), jnp.float32)
        return 0
    lax.fori_loop(0, n_words // 16, body, 0, unroll=4)


def _make_deg_kernel(n_pad, acc_rows, idx_rows_per_tile):
    rows_all = idx_rows_per_tile * NC * NS
    stripe = acc_rows // NS

    @functools.partial(
        pl.kernel,
        out_type=jax.ShapeDtypeStruct((NC, n_pad), jnp.float32),
        mesh=_mesh(),
        scratch_types=[
            pltpu.VMEM_SHARED((acc_rows,), jnp.float32),
            pltpu.VMEM((idx_rows_per_tile, LANES), jnp.int32),
            pltpu.VMEM((LANES,), jnp.float32),
            pltpu.VMEM((stripe,), jnp.float32),
        ],
    )
    def deg_kernel(dst_hbm, out_hbm, deg_sh, dstv, ones, zbuf):
        c = lax.axis_index("c")
        s = lax.axis_index("s")
        wid = c * NS + s
        _zero_fill(zbuf, stripe)
        for k in range(LANES // 16):
            ones[pl.ds(k * 16, 16)] = jnp.ones((16,), jnp.float32)
        pltpu.sync_copy(zbuf, deg_sh.at[pl.ds(s * stripe, stripe)])
        plsc.subcore_barrier()
        pltpu.sync_copy(dst_hbm.at[pl.ds(wid * idx_rows_per_tile,
                                         idx_rows_per_tile)], dstv)
        def chunk(j, _):
            pltpu.sync_copy(ones, deg_sh.at[dstv.at[j]], add=True)
            return 0
        lax.fori_loop(0, idx_rows_per_tile, chunk, 0)
        plsc.subcore_barrier()
        @pl.when(s == 0)
        def _():
            pltpu.sync_copy(deg_sh.at[pl.ds(0, n_pad)], out_hbm.at[c])

    return deg_kernel


def _make_conv_kernel(n_pad, d, acc_rows, idx_rows_per_tile):
    stripe = acc_rows // NS
    out_stripe = n_pad // NS

    @functools.partial(
        pl.kernel,
        out_type=jax.ShapeDtypeStruct((NC, n_pad, d), jnp.float32),
        mesh=_mesh(),
        scratch_types=[
            pltpu.VMEM_SHARED((acc_rows, d), jnp.float32),
            pltpu.VMEM((idx_rows_per_tile, LANES), jnp.int32),
            pltpu.VMEM((idx_rows_per_tile, LANES), jnp.int32),
            pltpu.VMEM((LANES, d), jnp.float32),
            pltpu.SemaphoreType.DMA,
        ],
    )
    def conv_kernel(h_hbm, src_hbm, dst_hbm, out_hbm,
                    acc_sh, srcv, dstv, rows, sem):
        c = lax.axis_index("c")
        s = lax.axis_index("s")
        wid = c * NS + s
        def zrow(k, _):
            rows[k // (d // 16), pl.ds((k % (d // 16)) * 16, 16)] = (
                jnp.zeros((16,), jnp.float32))
            return 0
        lax.fori_loop(0, LANES * (d // 16), zrow, 0, unroll=4)
        for r in range(stripe // LANES):
            pltpu.sync_copy(rows, acc_sh.at[pl.ds(s * stripe + r * LANES,
                                                  LANES)])
        plsc.subcore_barrier()
        pltpu.sync_copy(src_hbm.at[pl.ds(wid * idx_rows_per_tile,
                                         idx_rows_per_tile)], srcv)
        pltpu.sync_copy(dst_hbm.at[pl.ds(wid * idx_rows_per_tile,
                                         idx_rows_per_tile)], dstv)
        def chunk(j, _):
            pltpu.async_copy(h_hbm.at[srcv.at[j]], rows, sem).wait()
            pltpu.sync_copy(rows, acc_sh.at[dstv.at[j]], add=True)
            return 0
        lax.fori_loop(0, idx_rows_per_tile, chunk, 0)
        plsc.subcore_barrier()
        pltpu.sync_copy(acc_sh.at[pl.ds(s * out_stripe, out_stripe)],
                        out_hbm.at[c, pl.ds(s * out_stripe, out_stripe)])

    return conv_kernel


def _tc1_body(n, n_pad, deg_ref, x_ref, w_ref, dinv_ref, hp_ref):
    deg = deg_ref[0] + deg_ref[1] + 1.0
    dinv = lax.rsqrt(deg)
    dinv_ref[...] = dinv
    h = jnp.dot(x_ref[...], w_ref[...], preferred_element_type=jnp.float32)
    hp_ref[0:n, :] = h * dinv[0:n]
    hp_ref[n:n_pad, :] = jnp.zeros((n_pad - n, h.shape[1]), jnp.float32)


def _tc2_body(s_ref, hp_ref, dinv_ref, b_ref, g_ref, be_ref, w_ref,
              x1_ref, h2p_ref):
    dinv = dinv_ref[...]
    z = dinv * (s_ref[0] + s_ref[1] + hp_ref[...]) + b_ref[...]
    mu = jnp.mean(z, axis=-1, keepdims=True)
    var = jnp.mean((z - mu) ** 2, axis=-1, keepdims=True)
    zn = (z - mu) * lax.rsqrt(var + 1e-5) * g_ref[...] + be_ref[...]
    x1 = jnp.maximum(zn, 0.0)
    x1_ref[...] = x1
    h2p_ref[...] = jnp.dot(x1, w_ref[...],
                           preferred_element_type=jnp.float32) * dinv


def _tc3_body(n, s_ref, hp_ref, dinv_ref, b_ref, g_ref, be_ref, x1_ref,
              wf_ref, bf_ref, out_ref):
    dinv = dinv_ref[...]
    z = dinv * (s_ref[0] + s_ref[1] + hp_ref[...]) + b_ref[...]
    mu = jnp.mean(z, axis=-1, keepdims=True)
    var = jnp.mean((z - mu) ** 2, axis=-1, keepdims=True)
    zn = (z - mu) * lax.rsqrt(var + 1e-5) * g_ref[...] + be_ref[...]
    x2 = jnp.maximum(zn, 0.0) + x1_ref[...]
    out = jnp.dot(x2, wf_ref[...], preferred_element_type=jnp.float32)
    out_ref[...] = out[0:n, :] + bf_ref[...]


def kernel(x, edge_index, W1, b1, g1, be1, W2, b2, g2, be2, Wf, bf):
    n, d = x.shape
    h = W1.shape[1]
    e = edge_index.shape[1]

    n_pad = ((n + 1 + 127) // 128) * 128
    acc_rows = ((n_pad + NS * LANES - 1) // (NS * LANES)) * NS * LANES
    epb = NC * NS * LANES * 8
    e_pad = ((e + epb - 1) // epb) * epb
    idx_rows_per_tile = e_pad // (NC * NS * LANES)

    src = edge_index[0]
    dst = edge_index[1]
    pad = jnp.full((e_pad - e,), n, dtype=edge_index.dtype)
    srcr = jnp.concatenate([src, pad]).reshape(e_pad // LANES, LANES)
    dstr = jnp.concatenate([dst, pad]).reshape(e_pad // LANES, LANES)

    deg_kernel = _make_deg_kernel(n_pad, acc_rows, idx_rows_per_tile)
    conv_kernel = _make_conv_kernel(n_pad, d, acc_rows, idx_rows_per_tile)

    deg2 = deg_kernel(dstr)
    deg3 = deg2.reshape(NC, n_pad, 1)

    dinv, h1p = pl.pallas_call(
        functools.partial(_tc1_body, n, n_pad),
        out_shape=(jax.ShapeDtypeStruct((n_pad, 1), jnp.float32),
                   jax.ShapeDtypeStruct((n_pad, h), jnp.float32)),
    )(deg3, x, W1)

    s1 = conv_kernel(h1p, srcr, dstr)

    x1, h2p = pl.pallas_call(
        _tc2_body,
        out_shape=(jax.ShapeDtypeStruct((n_pad, h), jnp.float32),
                   jax.ShapeDtypeStruct((n_pad, h), jnp.float32)),
    )(s1, h1p, dinv, b1, g1, be1, W2)

    s2 = conv_kernel(h2p, srcr, dstr)

    out = pl.pallas_call(
        functools.partial(_tc3_body, n),
        out_shape=jax.ShapeDtypeStruct((n, Wf.shape[1]), jnp.float32),
    )(s2, h2p, dinv, b2, g2, be2, x1, Wf, bf)
    return out
